# baseline (device time: 75067 ns/iter reference)
import jax
import jax.numpy as jnp
from jax import lax
from jax.experimental import pallas as pl
from jax.experimental.pallas import tpu as pltpu

N_DEV = 8
N_TOK = 1024
D_IN = 512
D_OUT = 1024
N_EXP = 32
E_LOCAL = N_EXP // N_DEV
CHUNK = N_TOK // N_DEV


def kernel(x, router_W, route_idx, expert_W):
    def body(x_ref, rw_ref, idx_ref, ew_ref, out_ref,
             acc_ref, comm_ref, send_sems, recv_sems):
        my = lax.axis_index("i")
        left = lax.rem(my - 1 + N_DEV, N_DEV)
        right = lax.rem(my + 1, N_DEV)

        xf = x_ref[...]
        scores = jnp.dot(xf, rw_ref[...],
                         preferred_element_type=jnp.float32)
        m = jnp.max(scores, axis=-1, keepdims=True)
        p = jnp.exp(scores - m)
        e0 = idx_ref[:, 0:1]
        e1 = idx_ref[:, 1:2]
        iota = lax.broadcasted_iota(jnp.int32, (N_TOK, N_EXP), 1)
        g0 = jnp.sum(jnp.where(iota == e0, p, 0.0), axis=1, keepdims=True)
        g1 = jnp.sum(jnp.where(iota == e1, p, 0.0), axis=1, keepdims=True)
        gs = g0 + g1

        acc_ref[...] = jnp.zeros((N_TOK, D_OUT), jnp.float32)
        for k in range(E_LOCAL):
            eg = my * E_LOCAL + k
            gk = (jnp.where(e0 == eg, g0, 0.0)
                  + jnp.where(e1 == eg, g1, 0.0)) / gs
            xg = (xf * gk).astype(jnp.bfloat16)
            acc_ref[...] += jnp.dot(
                xg, ew_ref[k].astype(jnp.bfloat16),
                preferred_element_type=jnp.float32)

        barrier_sem = pltpu.get_barrier_semaphore()
        for nbr in (left, right):
            pl.semaphore_signal(barrier_sem, inc=1, device_id=(nbr,),
                                device_id_type=pl.DeviceIdType.MESH)
        pl.semaphore_wait(barrier_sem, 2)

        c0 = lax.rem(my - 1 + N_DEV, N_DEV)
        comm_ref[0, :, :] = acc_ref[pl.ds(c0 * CHUNK, CHUNK), :]
        for s in range(N_DEV - 1):
            send_slot = s % 2
            recv_slot = (s + 1) % 2
            rdma = pltpu.make_async_remote_copy(
                src_ref=comm_ref.at[send_slot],
                dst_ref=comm_ref.at[recv_slot],
                send_sem=send_sems.at[send_slot],
                recv_sem=recv_sems.at[recv_slot],
                device_id=(right,),
                device_id_type=pl.DeviceIdType.MESH,
            )
            rdma.start()
            rdma.wait()
            c = lax.rem(my - s - 2 + 2 * N_DEV, N_DEV)
            local = acc_ref[pl.ds(c * CHUNK, CHUNK), :]
            if s < N_DEV - 2:
                comm_ref[recv_slot, :, :] += local
            else:
                out_ref[...] = comm_ref[recv_slot, :, :] + local

    return pl.pallas_call(
        body,
        out_shape=jax.ShapeDtypeStruct((CHUNK, D_OUT), jnp.float32),
        in_specs=[
            pl.BlockSpec(memory_space=pltpu.VMEM),
            pl.BlockSpec(memory_space=pltpu.VMEM),
            pl.BlockSpec(memory_space=pltpu.VMEM),
            pl.BlockSpec(memory_space=pltpu.VMEM),
        ],
        out_specs=pl.BlockSpec(memory_space=pltpu.VMEM),
        scratch_shapes=[
            pltpu.VMEM((N_TOK, D_OUT), jnp.float32),
            pltpu.VMEM((2, CHUNK, D_OUT), jnp.float32),
            pltpu.SemaphoreType.DMA((2,)),
            pltpu.SemaphoreType.DMA((2,)),
        ],
        compiler_params=pltpu.CompilerParams(collective_id=0),
    )(x, router_W, route_idx, expert_W)


# device time: 39993 ns/iter; 1.8770x vs baseline; 1.8770x over previous
import jax
import jax.numpy as jnp
from jax import lax
from jax.experimental import pallas as pl
from jax.experimental.pallas import tpu as pltpu

N_DEV = 8
N_TOK = 1024
D_IN = 512
D_OUT = 1024
N_EXP = 32
E_LOCAL = N_EXP // N_DEV
CHUNK = N_TOK // N_DEV


def kernel(x, router_W, route_idx, expert_W):
    def body(x_ref, rw_ref, idx_ref, ew_ref, out_ref,
             acc_ref, send_ref, recv_ref, send_sems, recv_sems):
        my = lax.axis_index("i")

        xf = x_ref[...]
        scores = jnp.dot(xf, rw_ref[...],
                         preferred_element_type=jnp.float32)
        m = jnp.max(scores, axis=-1, keepdims=True)
        p = jnp.exp(scores - m)
        e0 = idx_ref[:, 0:1]
        e1 = idx_ref[:, 1:2]
        iota = lax.broadcasted_iota(jnp.int32, (N_TOK, N_EXP), 1)
        g0 = jnp.sum(jnp.where(iota == e0, p, 0.0), axis=1, keepdims=True)
        g1 = jnp.sum(jnp.where(iota == e1, p, 0.0), axis=1, keepdims=True)
        gs = g0 + g1

        acc_ref[...] = jnp.zeros((N_TOK, D_OUT), jnp.float32)
        for k in range(E_LOCAL):
            eg = my * E_LOCAL + k
            gk = (jnp.where(e0 == eg, g0, 0.0)
                  + jnp.where(e1 == eg, g1, 0.0)) / gs
            xg = (xf * gk).astype(jnp.bfloat16)
            acc_ref[...] += jnp.dot(
                xg, ew_ref[k].astype(jnp.bfloat16),
                preferred_element_type=jnp.float32)

        barrier_sem = pltpu.get_barrier_semaphore()
        for o in range(1, N_DEV):
            pl.semaphore_signal(barrier_sem, inc=1,
                                device_id=(lax.rem(my + o, N_DEV),),
                                device_id_type=pl.DeviceIdType.MESH)
        pl.semaphore_wait(barrier_sem, N_DEV - 1)

        rdmas = []
        for o in range(1, N_DEV):
            j = lax.rem(my + o, N_DEV)
            send_ref[o - 1, :, :] = acc_ref[
                pl.ds(j * CHUNK, CHUNK), :].astype(jnp.bfloat16)
            rdma = pltpu.make_async_remote_copy(
                src_ref=send_ref.at[o - 1],
                dst_ref=recv_ref.at[o - 1],
                send_sem=send_sems.at[o - 1],
                recv_sem=recv_sems.at[o - 1],
                device_id=(j,),
                device_id_type=pl.DeviceIdType.MESH,
            )
            rdma.start()
            rdmas.append(rdma)

        out_ref[...] = acc_ref[pl.ds(my * CHUNK, CHUNK), :]
        for k in range(N_DEV - 1):
            rdmas[k].wait_recv()
            out_ref[...] += recv_ref[k].astype(jnp.float32)
        for k in range(N_DEV - 1):
            rdmas[k].wait_send()

    return pl.pallas_call(
        body,
        out_shape=jax.ShapeDtypeStruct((CHUNK, D_OUT), jnp.float32),
        in_specs=[
            pl.BlockSpec(memory_space=pltpu.VMEM),
            pl.BlockSpec(memory_space=pltpu.VMEM),
            pl.BlockSpec(memory_space=pltpu.VMEM),
            pl.BlockSpec(memory_space=pltpu.VMEM),
        ],
        out_specs=pl.BlockSpec(memory_space=pltpu.VMEM),
        scratch_shapes=[
            pltpu.VMEM((N_TOK, D_OUT), jnp.float32),
            pltpu.VMEM((N_DEV - 1, CHUNK, D_OUT), jnp.bfloat16),
            pltpu.VMEM((N_DEV - 1, CHUNK, D_OUT), jnp.bfloat16),
            pltpu.SemaphoreType.DMA((N_DEV - 1,)),
            pltpu.SemaphoreType.DMA((N_DEV - 1,)),
        ],
        compiler_params=pltpu.CompilerParams(collective_id=0),
    )(x, router_W, route_idx, expert_W)


# device time: 32436 ns/iter; 2.3143x vs baseline; 1.2330x over previous
import jax
import jax.numpy as jnp
from jax import lax
from jax.experimental import pallas as pl
from jax.experimental.pallas import tpu as pltpu

N_DEV = 8
N_TOK = 1024
D_IN = 512
D_OUT = 1024
N_EXP = 32
E_LOCAL = N_EXP // N_DEV
CHUNK = N_TOK // N_DEV


def kernel(x, router_W, route_idx, expert_W):
    def body(x_ref, rw_ref, idx_ref, ew_ref, out_ref,
             xg_ref, ewb_ref, send_ref, recv_ref, send_sems, recv_sems):
        my = lax.axis_index("i")

        barrier_sem = pltpu.get_barrier_semaphore()
        for o in range(1, N_DEV):
            pl.semaphore_signal(barrier_sem, inc=1,
                                device_id=(lax.rem(my + o, N_DEV),),
                                device_id_type=pl.DeviceIdType.MESH)
        pl.semaphore_wait(barrier_sem, N_DEV - 1)

        xf = x_ref[...]
        scores = jnp.dot(xf, rw_ref[...],
                         preferred_element_type=jnp.float32)
        m = jnp.max(scores, axis=-1, keepdims=True)
        p = jnp.exp(scores - m)
        e0 = idx_ref[:, 0:1]
        e1 = idx_ref[:, 1:2]
        iota = lax.broadcasted_iota(jnp.int32, (N_TOK, N_EXP), 1)
        g0 = jnp.sum(jnp.where(iota == e0, p, 0.0), axis=1, keepdims=True)
        g1 = jnp.sum(jnp.where(iota == e1, p, 0.0), axis=1, keepdims=True)
        gs = g0 + g1

        ewb_ref[...] = jnp.reshape(
            ew_ref[...].astype(jnp.bfloat16), (E_LOCAL * D_IN, D_OUT))
        for k in range(E_LOCAL):
            eg = my * E_LOCAL + k
            gk = (jnp.where(e0 == eg, g0, 0.0)
                  + jnp.where(e1 == eg, g1, 0.0)) / gs
            xg_ref[:, k * D_IN:(k + 1) * D_IN] = (xf * gk).astype(jnp.bfloat16)

        rdmas = []
        for o in range(1, N_DEV):
            j = lax.rem(my + o, N_DEV)
            pc = jnp.dot(xg_ref[pl.ds(j * CHUNK, CHUNK), :], ewb_ref[...],
                         preferred_element_type=jnp.float32)
            send_ref[o - 1, :, :] = pc.astype(jnp.bfloat16)
            rdma = pltpu.make_async_remote_copy(
                src_ref=send_ref.at[o - 1],
                dst_ref=recv_ref.at[o - 1],
                send_sem=send_sems.at[o - 1],
                recv_sem=recv_sems.at[o - 1],
                device_id=(j,),
                device_id_type=pl.DeviceIdType.MESH,
            )
            rdma.start()
            rdmas.append(rdma)

        out_ref[...] = jnp.dot(
            xg_ref[pl.ds(my * CHUNK, CHUNK), :], ewb_ref[...],
            preferred_element_type=jnp.float32)

        for k in range(N_DEV - 1):
            rdmas[k].wait_recv()
            out_ref[...] += recv_ref[k].astype(jnp.float32)
        for k in range(N_DEV - 1):
            rdmas[k].wait_send()

    return pl.pallas_call(
        body,
        out_shape=jax.ShapeDtypeStruct((CHUNK, D_OUT), jnp.float32),
        in_specs=[
            pl.BlockSpec(memory_space=pltpu.VMEM),
            pl.BlockSpec(memory_space=pltpu.VMEM),
            pl.BlockSpec(memory_space=pltpu.VMEM),
            pl.BlockSpec(memory_space=pltpu.VMEM),
        ],
        out_specs=pl.BlockSpec(memory_space=pltpu.VMEM),
        scratch_shapes=[
            pltpu.VMEM((N_TOK, E_LOCAL * D_IN), jnp.bfloat16),
            pltpu.VMEM((E_LOCAL * D_IN, D_OUT), jnp.bfloat16),
            pltpu.VMEM((N_DEV - 1, CHUNK, D_OUT), jnp.bfloat16),
            pltpu.VMEM((N_DEV - 1, CHUNK, D_OUT), jnp.bfloat16),
            pltpu.SemaphoreType.DMA((N_DEV - 1,)),
            pltpu.SemaphoreType.DMA((N_DEV - 1,)),
        ],
        compiler_params=pltpu.CompilerParams(collective_id=0),
    )(x, router_W, route_idx, expert_W)


# device time: 25423 ns/iter; 2.9527x vs baseline; 1.2759x over previous
import jax
import jax.numpy as jnp
from jax import lax
from jax.experimental import pallas as pl
from jax.experimental.pallas import tpu as pltpu

N_DEV = 8
N_TOK = 1024
D_IN = 512
D_OUT = 1024
N_EXP = 32
E_LOCAL = N_EXP // N_DEV
CHUNK = N_TOK // N_DEV
BLK = 32
NBLK = CHUNK // BLK

GROUPS = ((0,), (1, 2), (3, 4), (5, 6, 7))


def kernel(x, router_W, route_idx, expert_W):
    def body(x_ref, rw_ref, idx_ref, ew_ref, out_ref,
             probs_ref, xg_ref, ewb_ref, send_ref, recv_ref,
             send_sems, recv_sems):
        my = lax.axis_index("i")

        recv_ref[...] = jnp.zeros_like(recv_ref)

        barrier_sem = pltpu.get_barrier_semaphore()
        for o in range(1, N_DEV):
            pl.semaphore_signal(barrier_sem, inc=1,
                                device_id=(lax.rem(my + o, N_DEV),),
                                device_id_type=pl.DeviceIdType.MESH)

        scores = jnp.dot(x_ref[...], rw_ref[...],
                         preferred_element_type=jnp.float32)
        m = jnp.max(scores, axis=-1, keepdims=True)
        probs_ref[...] = jnp.exp(scores - m)

        ewb_ref[...] = jnp.reshape(
            ew_ref[...].astype(jnp.bfloat16), (E_LOCAL * D_IN, D_OUT))

        iota_c = lax.broadcasted_iota(jnp.int32, (CHUNK, N_EXP), 1)
        iota_r = lax.broadcasted_iota(jnp.int32, (CHUNK, CHUNK), 0)
        iota_l = lax.broadcasted_iota(jnp.int32, (CHUNK, CHUNK), 1)
        lowtri = (iota_l < iota_r).astype(jnp.bfloat16)

        def routed(e0c, e1c, dev):
            lo = dev * E_LOCAL
            hi = lo + E_LOCAL
            return (((e0c >= lo) & (e0c < hi))
                    | ((e1c >= lo) & (e1c < hi)))

        def onehot_and_count(v):
            vb = v.astype(jnp.bfloat16)
            c = jnp.dot(lowtri, vb, preferred_element_type=jnp.float32)
            ci = c.astype(jnp.int32)
            oh = jnp.where((iota_l == ci) & v, 1.0, 0.0).astype(jnp.bfloat16)
            n = jnp.sum(v.astype(jnp.int32))
            return oh, n

        def pack_block(b, j):
            xc = x_ref[pl.ds(j * CHUNK, CHUNK), :]
            pc = probs_ref[pl.ds(j * CHUNK, CHUNK), :]
            e0c = idx_ref[pl.ds(j * CHUNK, CHUNK), 0:1]
            e1c = idx_ref[pl.ds(j * CHUNK, CHUNK), 1:2]
            g0c = jnp.sum(jnp.where(iota_c == e0c, pc, 0.0),
                          axis=1, keepdims=True)
            g1c = jnp.sum(jnp.where(iota_c == e1c, pc, 0.0),
                          axis=1, keepdims=True)
            gsc = g0c + g1c
            for k in range(E_LOCAL):
                eg = my * E_LOCAL + k
                gkc = (jnp.where(e0c == eg, g0c, 0.0)
                       + jnp.where(e1c == eg, g1c, 0.0)) / gsc
                xg_ref[b * CHUNK:(b + 1) * CHUNK,
                       k * D_IN:(k + 1) * D_IN] = (xc * gkc).astype(jnp.bfloat16)
            if b < N_DEV - 1:
                return onehot_and_count(routed(e0c, e1c, my))
            return None, None

        def block_rdma(b, mblk, dest):
            return pltpu.make_async_remote_copy(
                src_ref=send_ref.at[b, pl.ds(mblk * BLK, BLK)],
                dst_ref=recv_ref.at[b, pl.ds(mblk * BLK, BLK)],
                send_sem=send_sems.at[b, mblk],
                recv_sem=recv_sems.at[b, mblk],
                device_id=(dest,),
                device_id_type=pl.DeviceIdType.MESH,
            )

        n_send = [None] * (N_DEV - 1)
        waited_barrier = False
        for grp in GROUPS:
            ohs = {}
            for b in grp:
                j = lax.rem(my + b + 1, N_DEV) if b < 7 else my
                ohs[b] = pack_block(b, j)
            r0, r1 = grp[0] * CHUNK, (grp[-1] + 1) * CHUNK
            pcs = jnp.dot(xg_ref[r0:r1, :], ewb_ref[...],
                          preferred_element_type=jnp.float32)
            if not waited_barrier:
                pl.semaphore_wait(barrier_sem, N_DEV - 1)
                waited_barrier = True
            for b in grp:
                rows = pcs[(b - grp[0]) * CHUNK:(b - grp[0] + 1) * CHUNK, :]
                if b < 7:
                    oh, n = ohs[b]
                    n_send[b] = n
                    compact = lax.dot_general(
                        oh, rows.astype(jnp.bfloat16),
                        ((( 0,), (0,)), ((), ())),
                        preferred_element_type=jnp.float32)
                    send_ref[b, :, :] = compact.astype(jnp.bfloat16)
                    dest = lax.rem(my + b + 1, N_DEV)
                    for mblk in range(NBLK):
                        @pl.when(mblk * BLK < n)
                        def _(b=b, mblk=mblk, dest=dest):
                            block_rdma(b, mblk, dest).start()
                else:
                    out_ref[...] = rows

        e0m = idx_ref[pl.ds(my * CHUNK, CHUNK), 0:1]
        e1m = idx_ref[pl.ds(my * CHUNK, CHUNK), 1:2]
        oh_recv = []
        for b in range(N_DEV - 1):
            src = lax.rem(my - b - 1 + N_DEV, N_DEV)
            oh, n = onehot_and_count(routed(e0m, e1m, src))
            oh_recv.append(oh)
            for mblk in range(NBLK):
                @pl.when(mblk * BLK < n)
                def _(b=b, mblk=mblk, src=src):
                    block_rdma(b, mblk, src).wait_recv()

        oh_all = jnp.concatenate(oh_recv, axis=1)
        recv_all = jnp.reshape(recv_ref[...], ((N_DEV - 1) * CHUNK, D_OUT))
        out_ref[...] += jnp.dot(oh_all, recv_all,
                                preferred_element_type=jnp.float32)

        for b in range(N_DEV - 1):
            dest = lax.rem(my + b + 1, N_DEV)
            for mblk in range(NBLK):
                @pl.when(mblk * BLK < n_send[b])
                def _(b=b, mblk=mblk, dest=dest):
                    block_rdma(b, mblk, dest).wait_send()

    return pl.pallas_call(
        body,
        out_shape=jax.ShapeDtypeStruct((CHUNK, D_OUT), jnp.float32),
        in_specs=[pl.BlockSpec(memory_space=pltpu.VMEM)] * 4,
        out_specs=pl.BlockSpec(memory_space=pltpu.VMEM),
        scratch_shapes=[
            pltpu.VMEM((N_TOK, N_EXP), jnp.float32),
            pltpu.VMEM((N_TOK, E_LOCAL * D_IN), jnp.bfloat16),
            pltpu.VMEM((E_LOCAL * D_IN, D_OUT), jnp.bfloat16),
            pltpu.VMEM((N_DEV - 1, CHUNK, D_OUT), jnp.bfloat16),
            pltpu.VMEM((N_DEV - 1, CHUNK, D_OUT), jnp.bfloat16),
            pltpu.SemaphoreType.DMA((N_DEV - 1, NBLK)),
            pltpu.SemaphoreType.DMA((N_DEV - 1, NBLK)),
        ],
        compiler_params=pltpu.CompilerParams(collective_id=0),
    )(x, router_W, route_idx, expert_W)
